# baseline (speedup 1.0000x reference)
import functools

import jax
import jax.numpy as jnp
from jax import lax
from jax.experimental import pallas as pl
from jax.experimental.pallas import tpu as pltpu

D_MODEL = 768
N_EXPERTS = 64
K = 8
N_TOKENS = 32768
BLOCK = 4096
AUX_COEF = 0.001


def _router_body(x_ref, w_ref, idx_ref, wgt_ref, aux_ref, acc_ref):
    step = pl.program_id(0)
    nsteps = pl.num_programs(0)

    x = x_ref[...]
    w = w_ref[...]
    logits = lax.dot_general(
        w, x, (((1,), (1,)), ((), ())), preferred_element_type=jnp.float32
    )

    colmax = jnp.max(logits, axis=0, keepdims=True)
    e = jnp.exp(logits - colmax)
    colsum = jnp.sum(e, axis=0, keepdims=True)
    probs = e * (1.0 / colsum)

    @pl.when(step == 0)
    def _():
        acc_ref[...] = jnp.zeros_like(acc_ref)

    acc_ref[...] += jnp.sum(probs, axis=1, keepdims=True)

    sublanes = lax.broadcasted_iota(jnp.int32, probs.shape, 0)
    work = probs
    vals = []
    idxs = []
    for _ in range(K):
        m = jnp.max(work, axis=0, keepdims=True)
        hit = work == m
        idx = jnp.min(jnp.where(hit, sublanes, N_EXPERTS), axis=0, keepdims=True)
        vals.append(m)
        idxs.append(idx)
        work = jnp.where(sublanes == idx, -1.0, work)

    topv = jnp.concatenate(vals, axis=0)
    topi = jnp.concatenate(idxs, axis=0)
    wgt_ref[...] = (topv / jnp.sum(topv, axis=0, keepdims=True)).T
    idx_ref[...] = topi.T

    @pl.when(step == nsteps - 1)
    def _():
        avg = acc_ref[...] * (1.0 / N_TOKENS)
        aux_ref[...] = jnp.sum(avg * avg) * (N_EXPERTS * AUX_COEF) * jnp.ones_like(
            aux_ref
        )


@jax.jit
def _router(hidden_states, W):
    nblocks = N_TOKENS // BLOCK
    out_shapes = (
        jax.ShapeDtypeStruct((N_TOKENS, K), jnp.int32),
        jax.ShapeDtypeStruct((N_TOKENS, K), jnp.float32),
        jax.ShapeDtypeStruct((1, 1), jnp.float32),
    )
    idx, wgt, aux = pl.pallas_call(
        _router_body,
        grid=(nblocks,),
        in_specs=[
            pl.BlockSpec((BLOCK, D_MODEL), lambda i: (i, 0)),
            pl.BlockSpec((N_EXPERTS, D_MODEL), lambda i: (0, 0)),
        ],
        out_specs=(
            pl.BlockSpec((BLOCK, K), lambda i: (i, 0)),
            pl.BlockSpec((BLOCK, K), lambda i: (i, 0)),
            pl.BlockSpec((1, 1), lambda i: (0, 0)),
        ),
        out_shape=out_shapes,
        scratch_shapes=[pltpu.VMEM((N_EXPERTS, 1), jnp.float32)],
    )(hidden_states, W)
    return idx, wgt, aux[0, 0]


def kernel(hidden_states, W):
    return _router(hidden_states, W)


# speedup vs baseline: 1.0177x; 1.0177x over previous
import jax
import jax.numpy as jnp
from jax import lax
from jax.experimental import pallas as pl
from jax.experimental.pallas import tpu as pltpu

D_MODEL = 768
N_EXPERTS = 64
K = 8
N_TOKENS = 32768
BLOCK = 4096
AUX_COEF = 0.001


def _router_body(x_ref, w_ref, idx_ref, wgt_ref, aux_ref, acc_ref):
    step = pl.program_id(0)
    nsteps = pl.num_programs(0)

    x = x_ref[...]
    w = w_ref[...]
    logits = lax.dot_general(
        w, x, (((1,), (1,)), ((), ())), preferred_element_type=jnp.float32
    )

    colmax = jnp.max(logits, axis=0, keepdims=True)
    e = jnp.exp(logits - colmax)
    colsum = jnp.sum(e, axis=0, keepdims=True)
    probs = e * (1.0 / colsum)

    @pl.when(step == 0)
    def _():
        acc_ref[...] = jnp.zeros_like(acc_ref)

    acc_ref[...] += jnp.sum(probs, axis=1, keepdims=True)

    sublanes = lax.broadcasted_iota(jnp.int32, probs.shape, 0).astype(jnp.float32)
    work = probs
    vals = []
    idxs = []
    for _ in range(K):
        m = jnp.max(work, axis=0, keepdims=True)
        hit = work == m
        idx = jnp.min(
            jnp.where(hit, sublanes, float(N_EXPERTS)), axis=0, keepdims=True
        )
        vals.append(m)
        idxs.append(idx)
        work = jnp.where(sublanes == idx, -1.0, work)

    topv = jnp.concatenate(vals, axis=0)
    topi = jnp.concatenate(idxs, axis=0)
    wgt_ref[...] = (topv / jnp.sum(topv, axis=0, keepdims=True)).T
    idx_ref[...] = topi.T.astype(jnp.int32)

    @pl.when(step == nsteps - 1)
    def _():
        avg = acc_ref[...] * (1.0 / N_TOKENS)
        aux_ref[...] = jnp.sum(avg * avg) * (N_EXPERTS * AUX_COEF) * jnp.ones_like(
            aux_ref
        )


@jax.jit
def _router(hidden_states, W):
    nblocks = N_TOKENS // BLOCK
    out_shapes = (
        jax.ShapeDtypeStruct((N_TOKENS, K), jnp.int32),
        jax.ShapeDtypeStruct((N_TOKENS, K), jnp.float32),
        jax.ShapeDtypeStruct((1, 1), jnp.float32),
    )
    idx, wgt, aux = pl.pallas_call(
        _router_body,
        grid=(nblocks,),
        in_specs=[
            pl.BlockSpec((BLOCK, D_MODEL), lambda i: (i, 0)),
            pl.BlockSpec((N_EXPERTS, D_MODEL), lambda i: (0, 0)),
        ],
        out_specs=(
            pl.BlockSpec((BLOCK, K), lambda i: (i, 0)),
            pl.BlockSpec((BLOCK, K), lambda i: (i, 0)),
            pl.BlockSpec((1, 1), lambda i: (0, 0)),
        ),
        out_shape=out_shapes,
        scratch_shapes=[pltpu.VMEM((N_EXPERTS, 1), jnp.float32)],
    )(hidden_states, W)
    return idx, wgt, aux[0, 0]


def kernel(hidden_states, W):
    return _router(hidden_states, W)
